# Initial kernel scaffold; baseline (speedup 1.0000x reference)
import jax
import jax.numpy as jnp
from jax.experimental import pallas as pl


def kernel(x, Wg, bg, W1, b1, W2, b2):
    raise NotImplementedError("write your pallas kernel here")


# speedup vs baseline: 11.2403x; 11.2403x over previous
import jax
import jax.numpy as jnp
from jax.experimental import pallas as pl

_EMBED = 256
_NUM_EXPERTS = 4
_D_FF = _EMBED * 4
_TILE = 1024


def _moe_body(x_ref, wg_ref, bg_ref, w1_ref, b1_ref, w2_ref, b2_ref, o_ref):
    x = x_ref[...]
    g = jnp.dot(x, wg_ref[...], preferred_element_type=jnp.float32)
    g = jax.nn.softmax(g + bg_ref[...], axis=-1)
    h = jnp.dot(x, w1_ref[...], preferred_element_type=jnp.float32)
    h = h + b1_ref[...]
    h = 0.5 * h * (1.0 + jax.lax.erf(h * 0.7071067811865476))
    hs = jnp.concatenate(
        [h[:, i * _D_FF:(i + 1) * _D_FF] * g[:, i:i + 1]
         for i in range(_NUM_EXPERTS)], axis=1)
    out = jnp.dot(hs, w2_ref[...], preferred_element_type=jnp.float32)
    out = out + jnp.dot(g, b2_ref[...], preferred_element_type=jnp.float32)
    o_ref[...] = out


def kernel(x, Wg, bg, W1, b1, W2, b2):
    B, S, E = x.shape
    n_tok = B * S
    x2d = x.reshape(n_tok, E)
    w1cat = W1.transpose(1, 0, 2).reshape(E, _NUM_EXPERTS * _D_FF)
    b1cat = b1.reshape(1, _NUM_EXPERTS * _D_FF)
    w2cat = W2.reshape(_NUM_EXPERTS * _D_FF, E)
    bg2d = bg.reshape(1, _NUM_EXPERTS)

    grid = (n_tok // _TILE,)
    out = pl.pallas_call(
        _moe_body,
        grid=grid,
        in_specs=[
            pl.BlockSpec((_TILE, E), lambda i: (i, 0)),
            pl.BlockSpec((E, _NUM_EXPERTS), lambda i: (0, 0)),
            pl.BlockSpec((1, _NUM_EXPERTS), lambda i: (0, 0)),
            pl.BlockSpec((E, _NUM_EXPERTS * _D_FF), lambda i: (0, 0)),
            pl.BlockSpec((1, _NUM_EXPERTS * _D_FF), lambda i: (0, 0)),
            pl.BlockSpec((_NUM_EXPERTS * _D_FF, E), lambda i: (0, 0)),
            pl.BlockSpec((_NUM_EXPERTS, E), lambda i: (0, 0)),
        ],
        out_specs=pl.BlockSpec((_TILE, E), lambda i: (i, 0)),
        out_shape=jax.ShapeDtypeStruct((n_tok, E), jnp.float32),
    )(x2d, Wg, bg2d, w1cat, b1cat, w2cat, b2)
    return out.reshape(B, S, E)
